# baseline (speedup 1.0000x reference)
import functools

import jax
import jax.numpy as jnp
from jax import lax
from jax.experimental import pallas as pl
from jax.experimental.pallas import tpu as pltpu
from jax.experimental.pallas import tpu_sc as plsc

_NC = 2
_NS = 16
_NW = _NC * _NS
_CHUNK = 128


@functools.partial(jax.jit, static_argnums=(2, 3))
def _gather_call(x, idx3, n_chunks, d):
    b_per_w = n_chunks * _CHUNK
    mesh = plsc.VectorSubcoreMesh(core_axis_name="c", subcore_axis_name="s")

    @functools.partial(
        pl.kernel,
        mesh=mesh,
        out_type=jax.ShapeDtypeStruct((_NW * b_per_w, d), jnp.float32),
        scratch_types=[
            pltpu.VMEM((n_chunks, _CHUNK), jnp.int32),
            pltpu.VMEM((_CHUNK, d), jnp.float32),
            pltpu.SemaphoreType.DMA,
        ],
        compiler_params=pltpu.CompilerParams(use_tc_tiling_on_sc=False),
    )
    def body(table_hbm, idx_hbm, out_hbm, idx_v, rows_v, sem):
        wid = lax.axis_index("s") * _NC + lax.axis_index("c")
        base = wid * b_per_w
        pltpu.sync_copy(idx_hbm.at[wid], idx_v)

        def step(j, carry):
            pltpu.async_copy(table_hbm.at[idx_v.at[j]], rows_v, sem).wait()
            pltpu.sync_copy(rows_v, out_hbm.at[pl.ds(base + j * _CHUNK, _CHUNK)])
            return carry

        lax.fori_loop(0, n_chunks, step, 0)

    return body(x, idx3)


def kernel(x, dim, index):
    v, d = x.shape
    b = index.shape[0]
    idx = index.astype(jnp.int32) + jnp.asarray(dim, jnp.int32)

    grain = _NW * _CHUNK
    b_pad = ((b + grain - 1) // grain) * grain
    if b_pad != b:
        idx = jnp.pad(idx, (0, b_pad - b))
    n_chunks = b_pad // grain
    idx3 = idx.reshape(_NW, n_chunks, _CHUNK)

    out = _gather_call(x, idx3, n_chunks, d)
    if b_pad != b:
        out = out[:b]
    return out


# speedup vs baseline: 1.0748x; 1.0748x over previous
import functools

import jax
import jax.numpy as jnp
from jax import lax
from jax.experimental import pallas as pl
from jax.experimental.pallas import tpu as pltpu
from jax.experimental.pallas import tpu_sc as plsc

_NC = 2
_NS = 16
_NW = _NC * _NS
_CHUNK = 128


_KC = 4
_SLAB = _KC * _CHUNK


@functools.partial(jax.jit, static_argnums=(2, 3))
def _gather_call(x, idx3, n_chunks, d):
    b_per_w = n_chunks * _CHUNK
    n_slabs = n_chunks // _KC
    mesh = plsc.VectorSubcoreMesh(core_axis_name="c", subcore_axis_name="s")

    @functools.partial(
        pl.kernel,
        mesh=mesh,
        out_type=jax.ShapeDtypeStruct((_NW * b_per_w, d), jnp.float32),
        scratch_types=[
            pltpu.VMEM((n_chunks, _CHUNK), jnp.int32),
            pltpu.VMEM((2, _SLAB, d), jnp.float32),
            pltpu.SemaphoreType.DMA,
            pltpu.SemaphoreType.DMA,
        ],
        compiler_params=pltpu.CompilerParams(use_tc_tiling_on_sc=False),
    )
    def body(table_hbm, idx_hbm, out_hbm, idx_v, rows_v, gsem, wsem):
        wid = lax.axis_index("s") * _NC + lax.axis_index("c")
        base = wid * b_per_w
        pltpu.sync_copy(idx_hbm.at[wid], idx_v)

        def out_slab(s):
            return out_hbm.at[pl.ds(base + s * _SLAB, _SLAB)]

        def fire(s, p):
            for c in range(_KC):
                pltpu.async_copy(
                    table_hbm.at[idx_v.at[s * _KC + c]],
                    rows_v.at[p, pl.ds(c * _CHUNK, _CHUNK)],
                    gsem,
                )

        def drain_gathers(p):
            for c in range(_KC):
                pltpu.make_async_copy(
                    table_hbm.at[idx_v.at[c]],
                    rows_v.at[p, pl.ds(c * _CHUNK, _CHUNK)],
                    gsem,
                ).wait()

        def wait_write(s, p):
            pltpu.make_async_copy(rows_v.at[p], out_slab(s), wsem).wait()

        def step(s, carry):
            p = s % 2
            @pl.when(s >= 2)
            def _():
                wait_write(s - 2, p)

            fire(s, p)
            drain_gathers(p)
            pltpu.async_copy(rows_v.at[p], out_slab(s), wsem)
            return carry

        lax.fori_loop(0, n_slabs, step, 0)
        wait_write(n_slabs - 2, (n_slabs - 2) % 2)
        wait_write(n_slabs - 1, (n_slabs - 1) % 2)

    return body(x, idx3)


def kernel(x, dim, index):
    v, d = x.shape
    b = index.shape[0]
    idx = index.astype(jnp.int32) + jnp.asarray(dim, jnp.int32)

    grain = _NW * _SLAB
    b_pad = ((b + grain - 1) // grain) * grain
    if b_pad != b:
        idx = jnp.pad(idx, (0, b_pad - b))
    n_chunks = b_pad // (_NW * _CHUNK)
    idx3 = idx.reshape(_NW, n_chunks, _CHUNK)

    out = _gather_call(x, idx3, n_chunks, d)
    if b_pad != b:
        out = out[:b]
    return out
